# Initial kernel scaffold; baseline (speedup 1.0000x reference)
import jax
import jax.numpy as jnp
from jax.experimental import pallas as pl


def kernel(x_in, emb):
    raise NotImplementedError("write your pallas kernel here")


# speedup vs baseline: 2.8563x; 2.8563x over previous
import functools

import jax
import jax.numpy as jnp
from jax.experimental import pallas as pl

NUM_EMB = 1024
IN_DIM = 64
BETA = 0.25


def _vq_kernel(x_ref, emb_ref, xq_ref, idx_ref, loss_ref):
    b = pl.program_id(0)
    x = x_ref[0]
    emb = emb_ref[...]

    a2 = jnp.sum(x * x, axis=0, keepdims=True)
    b2 = jnp.sum(emb * emb, axis=1, keepdims=True)
    m = jax.lax.dot_general(
        emb, x, (((1,), (0,)), ((), ())),
        preferred_element_type=jnp.float32)
    d2 = (a2 + b2) - 2.0 * m

    dmin = jnp.min(d2, axis=0, keepdims=True)
    iota = jax.lax.broadcasted_iota(jnp.int32, d2.shape, 0)
    idx = jnp.min(jnp.where(d2 == dmin, iota, NUM_EMB), axis=0)
    idx_ref[0, 0] = idx

    onehot = (iota == idx[None, :]).astype(jnp.float32)
    x_q = jax.lax.dot_general(
        emb, onehot, (((0,), (0,)), ((), ())),
        preferred_element_type=jnp.float32)

    diff = x_q - x
    partial = jnp.sum(diff * diff, keepdims=True).reshape(1, 1)
    @pl.when(b == 0)
    def _init():
        loss_ref[...] = jnp.zeros_like(loss_ref)
    loss_ref[...] += partial

    xq_ref[0] = x + (x_q - x)


@jax.jit
def kernel(x_in, emb):
    B, C, L = x_in.shape
    x_q, idxs3, loss_sum = pl.pallas_call(
        _vq_kernel,
        grid=(B,),
        in_specs=[
            pl.BlockSpec((1, C, L), lambda b: (b, 0, 0)),
            pl.BlockSpec((NUM_EMB, IN_DIM), lambda b: (0, 0)),
        ],
        out_specs=[
            pl.BlockSpec((1, C, L), lambda b: (b, 0, 0)),
            pl.BlockSpec((1, 1, L), lambda b: (b, 0, 0)),
            pl.BlockSpec((1, 1), lambda b: (0, 0)),
        ],
        out_shape=[
            jax.ShapeDtypeStruct((B, C, L), jnp.float32),
            jax.ShapeDtypeStruct((B, 1, L), jnp.int32),
            jax.ShapeDtypeStruct((1, 1), jnp.float32),
        ],
    )(x_in, emb)
    idxs = idxs3.reshape(B, L)
    mean_sq = loss_sum[0, 0] / (B * C * L)
    vq_loss = mean_sq + BETA * mean_sq
    return (x_q, idxs, vq_loss)


# speedup vs baseline: 2.9239x; 1.0237x over previous
import functools

import jax
import jax.numpy as jnp
from jax.experimental import pallas as pl
from jax.experimental.pallas import tpu as pltpu

NUM_EMB = 1024
IN_DIM = 64
BETA = 0.25


def _vq_kernel(x_ref, emb_ref, xq_ref, idx_ref, loss_ref, b2_ref):
    b = pl.program_id(0)
    x = x_ref[0]
    emb = emb_ref[...]

    @pl.when(b == 0)
    def _precompute():
        b2_ref[...] = jnp.sum(emb * emb, axis=1, keepdims=True)
        loss_ref[...] = jnp.zeros_like(loss_ref)

    a2 = jnp.sum(x * x, axis=0, keepdims=True)
    b2 = b2_ref[...]
    m2 = jax.lax.dot_general(
        emb + emb, x, (((1,), (0,)), ((), ())),
        preferred_element_type=jnp.float32)
    d2 = (a2 + b2) - m2

    dmin = jnp.min(d2, axis=0, keepdims=True)
    iota_col = jax.lax.broadcasted_iota(
        jnp.int32, (NUM_EMB, 1), 0).astype(jnp.float32)
    idx_f = jnp.min(jnp.where(d2 == dmin, iota_col, float(NUM_EMB)),
                    axis=0)
    idx_ref[0, 0] = idx_f.astype(jnp.int32)

    onehot = (iota_col == idx_f[None, :]).astype(jnp.float32)
    x_q = jax.lax.dot_general(
        emb, onehot, (((0,), (0,)), ((), ())),
        preferred_element_type=jnp.float32)

    diff = x_q - x
    partial = jnp.sum(diff * diff, keepdims=True).reshape(1, 1)
    loss_ref[...] += partial

    xq_ref[0] = x + (x_q - x)


@jax.jit
def kernel(x_in, emb):
    B, C, L = x_in.shape
    x_q, idxs3, loss_sum = pl.pallas_call(
        _vq_kernel,
        grid=(B,),
        in_specs=[
            pl.BlockSpec((1, C, L), lambda b: (b, 0, 0)),
            pl.BlockSpec((NUM_EMB, IN_DIM), lambda b: (0, 0)),
        ],
        out_specs=[
            pl.BlockSpec((1, C, L), lambda b: (b, 0, 0)),
            pl.BlockSpec((1, 1, L), lambda b: (b, 0, 0)),
            pl.BlockSpec((1, 1), lambda b: (0, 0)),
        ],
        out_shape=[
            jax.ShapeDtypeStruct((B, C, L), jnp.float32),
            jax.ShapeDtypeStruct((B, 1, L), jnp.int32),
            jax.ShapeDtypeStruct((1, 1), jnp.float32),
        ],
        scratch_shapes=[pltpu.VMEM((NUM_EMB, 1), jnp.float32)],
    )(x_in, emb)
    idxs = idxs3.reshape(B, L)
    mean_sq = loss_sum[0, 0] / (B * C * L)
    vq_loss = mean_sq + BETA * mean_sq
    return (x_q, idxs, vq_loss)


# speedup vs baseline: 3.1628x; 1.0817x over previous
import functools

import jax
import jax.numpy as jnp
from jax.experimental import pallas as pl
from jax.experimental.pallas import tpu as pltpu

NUM_EMB = 1024
IN_DIM = 64
BETA = 0.25
BB = 2


def _vq_kernel(x_ref, emb_ref, xq_ref, idx_ref, loss_ref, b2_ref):
    b = pl.program_id(0)
    emb = emb_ref[...]

    @pl.when(b == 0)
    def _precompute():
        b2_ref[...] = jnp.sum(emb * emb, axis=1, keepdims=True)
        loss_ref[...] = jnp.zeros_like(loss_ref)

    b2 = b2_ref[...]
    emb2 = emb + emb
    iota_col = jax.lax.broadcasted_iota(
        jnp.int32, (NUM_EMB, 1), 0).astype(jnp.float32)

    for i in range(BB):
        x = x_ref[i]
        a2 = jnp.sum(x * x, axis=0, keepdims=True)
        m2 = jax.lax.dot_general(
            emb2, x, (((1,), (0,)), ((), ())),
            preferred_element_type=jnp.float32)
        d2 = (a2 + b2) - m2

        dmin = jnp.min(d2, axis=0, keepdims=True)
        idx_f = jnp.min(jnp.where(d2 == dmin, iota_col, float(NUM_EMB)),
                        axis=0)
        idx_ref[0, i] = idx_f.astype(jnp.int32)

        onehot = (iota_col == idx_f[None, :]).astype(jnp.float32)
        x_q = jax.lax.dot_general(
            emb, onehot, (((0,), (0,)), ((), ())),
            preferred_element_type=jnp.float32)

        diff = x_q - x
        partial = jnp.sum(diff * diff, keepdims=True).reshape(1, 1)
        loss_ref[...] += partial

        xq_ref[i] = x + (x_q - x)


@jax.jit
def kernel(x_in, emb):
    B, C, L = x_in.shape
    x_q, idxs3, loss_sum = pl.pallas_call(
        _vq_kernel,
        grid=(B // BB,),
        in_specs=[
            pl.BlockSpec((BB, C, L), lambda b: (b, 0, 0)),
            pl.BlockSpec((NUM_EMB, IN_DIM), lambda b: (0, 0)),
        ],
        out_specs=[
            pl.BlockSpec((BB, C, L), lambda b: (b, 0, 0)),
            pl.BlockSpec((1, BB, L), lambda b: (b, 0, 0)),
            pl.BlockSpec((1, 1), lambda b: (0, 0)),
        ],
        out_shape=[
            jax.ShapeDtypeStruct((B, C, L), jnp.float32),
            jax.ShapeDtypeStruct((B // BB, BB, L), jnp.int32),
            jax.ShapeDtypeStruct((1, 1), jnp.float32),
        ],
        scratch_shapes=[pltpu.VMEM((NUM_EMB, 1), jnp.float32)],
    )(x_in, emb)
    idxs = idxs3.reshape(B, L)
    mean_sq = loss_sum[0, 0] / (B * C * L)
    vq_loss = mean_sq + BETA * mean_sq
    return (x_q, idxs, vq_loss)


# speedup vs baseline: 3.2149x; 1.0164x over previous
import functools

import jax
import jax.numpy as jnp
from jax.experimental import pallas as pl
from jax.experimental.pallas import tpu as pltpu

NUM_EMB = 1024
IN_DIM = 64
BETA = 0.25
BB = 4


def _vq_kernel(x_ref, emb_ref, xq_ref, idx_ref, loss_ref, b2_ref):
    b = pl.program_id(0)
    emb = emb_ref[...]

    @pl.when(b == 0)
    def _precompute():
        b2_ref[...] = jnp.sum(emb * emb, axis=1, keepdims=True)
        loss_ref[...] = jnp.zeros_like(loss_ref)

    b2 = b2_ref[...]
    emb2 = emb + emb
    iota_col = jax.lax.broadcasted_iota(
        jnp.int32, (NUM_EMB, 1), 0).astype(jnp.float32)

    for i in range(BB):
        x = x_ref[i]
        a2 = jnp.sum(x * x, axis=0, keepdims=True)
        m2 = jax.lax.dot_general(
            emb2, x, (((1,), (0,)), ((), ())),
            preferred_element_type=jnp.float32)
        d2 = (a2 + b2) - m2

        dmin = jnp.min(d2, axis=0, keepdims=True)
        idx_f = jnp.min(jnp.where(d2 == dmin, iota_col, float(NUM_EMB)),
                        axis=0)
        idx_ref[0, i] = idx_f.astype(jnp.int32)

        onehot = (iota_col == idx_f[None, :]).astype(jnp.float32)
        x_q = jax.lax.dot_general(
            emb, onehot, (((0,), (0,)), ((), ())),
            preferred_element_type=jnp.float32)

        diff = x_q - x
        partial = jnp.sum(diff * diff, keepdims=True).reshape(1, 1)
        loss_ref[...] += partial

        xq_ref[i] = x + (x_q - x)


@jax.jit
def kernel(x_in, emb):
    B, C, L = x_in.shape
    x_q, idxs3, loss_sum = pl.pallas_call(
        _vq_kernel,
        grid=(B // BB,),
        in_specs=[
            pl.BlockSpec((BB, C, L), lambda b: (b, 0, 0)),
            pl.BlockSpec((NUM_EMB, IN_DIM), lambda b: (0, 0)),
        ],
        out_specs=[
            pl.BlockSpec((BB, C, L), lambda b: (b, 0, 0)),
            pl.BlockSpec((1, BB, L), lambda b: (b, 0, 0)),
            pl.BlockSpec((1, 1), lambda b: (0, 0)),
        ],
        out_shape=[
            jax.ShapeDtypeStruct((B, C, L), jnp.float32),
            jax.ShapeDtypeStruct((B // BB, BB, L), jnp.int32),
            jax.ShapeDtypeStruct((1, 1), jnp.float32),
        ],
        scratch_shapes=[pltpu.VMEM((NUM_EMB, 1), jnp.float32)],
    )(x_in, emb)
    idxs = idxs3.reshape(B, L)
    mean_sq = loss_sum[0, 0] / (B * C * L)
    vq_loss = mean_sq + BETA * mean_sq
    return (x_q, idxs, vq_loss)


# speedup vs baseline: 3.3292x; 1.0356x over previous
import functools

import jax
import jax.numpy as jnp
from jax.experimental import pallas as pl
from jax.experimental.pallas import tpu as pltpu

NUM_EMB = 1024
IN_DIM = 64
BETA = 0.25
BB = 8


def _vq_kernel(x_ref, emb_ref, xq_ref, idx_ref, loss_ref, b2_ref):
    b = pl.program_id(0)
    emb = emb_ref[...]

    @pl.when(b == 0)
    def _precompute():
        b2_ref[...] = jnp.sum(emb * emb, axis=1, keepdims=True)
        loss_ref[...] = jnp.zeros_like(loss_ref)

    b2 = b2_ref[...]
    emb2 = emb + emb
    iota_col = jax.lax.broadcasted_iota(
        jnp.int32, (NUM_EMB, 1), 0).astype(jnp.float32)

    for i in range(BB):
        x = x_ref[i]
        a2 = jnp.sum(x * x, axis=0, keepdims=True)
        m2 = jax.lax.dot_general(
            emb2, x, (((1,), (0,)), ((), ())),
            preferred_element_type=jnp.float32)
        d2 = (a2 + b2) - m2

        dmin = jnp.min(d2, axis=0, keepdims=True)
        idx_f = jnp.min(jnp.where(d2 == dmin, iota_col, float(NUM_EMB)),
                        axis=0)
        idx_ref[0, i] = idx_f.astype(jnp.int32)

        onehot = (iota_col == idx_f[None, :]).astype(jnp.float32)
        x_q = jax.lax.dot_general(
            emb, onehot, (((0,), (0,)), ((), ())),
            preferred_element_type=jnp.float32)

        diff = x_q - x
        partial = jnp.sum(diff * diff, keepdims=True).reshape(1, 1)
        loss_ref[...] += partial

        xq_ref[i] = x + (x_q - x)


@jax.jit
def kernel(x_in, emb):
    B, C, L = x_in.shape
    x_q, idxs3, loss_sum = pl.pallas_call(
        _vq_kernel,
        grid=(B // BB,),
        in_specs=[
            pl.BlockSpec((BB, C, L), lambda b: (b, 0, 0)),
            pl.BlockSpec((NUM_EMB, IN_DIM), lambda b: (0, 0)),
        ],
        out_specs=[
            pl.BlockSpec((BB, C, L), lambda b: (b, 0, 0)),
            pl.BlockSpec((1, BB, L), lambda b: (b, 0, 0)),
            pl.BlockSpec((1, 1), lambda b: (0, 0)),
        ],
        out_shape=[
            jax.ShapeDtypeStruct((B, C, L), jnp.float32),
            jax.ShapeDtypeStruct((B // BB, BB, L), jnp.int32),
            jax.ShapeDtypeStruct((1, 1), jnp.float32),
        ],
        scratch_shapes=[pltpu.VMEM((NUM_EMB, 1), jnp.float32)],
    )(x_in, emb)
    idxs = idxs3.reshape(B, L)
    mean_sq = loss_sum[0, 0] / (B * C * L)
    vq_loss = mean_sq + BETA * mean_sq
    return (x_q, idxs, vq_loss)
